# baseline (speedup 1.0000x reference)
import functools

import jax
import jax.numpy as jnp
from jax import lax
from jax.experimental import pallas as pl
from jax.experimental.pallas import tpu as pltpu
from jax.experimental.pallas import tpu_sc as plsc

N = 10000
D = 128
B = 512
HID = 64
C = 2

NC = 2
NS = 16
NW = NC * NS
CHUNK = 320
SUB = 64
NSUB = CHUNK // SUB
R = 640
RPW = R // NS
OVERLAP = NW * CHUNK - N

NP = NW * CHUNK
NPR = NP // 128

def _sc_pool_body(x_hbm, batch_hbm, sums_hbm,
                  rows_v, idx_v, zsum_v, sums_sh, xsem, bsem):
    c = lax.axis_index("c")
    s = lax.axis_index("s")
    wid = c * NS + s
    base = jnp.minimum(wid * CHUNK, N - CHUNK)

    x_dma = pltpu.async_copy(x_hbm.at[pl.ds(base, CHUNK)], rows_v, xsem)
    b_dmas = [
        pltpu.async_copy(batch_hbm.at[pl.ds(base + j * SUB, SUB)],
                         idx_v.at[j], bsem)
        for j in range(NSUB)
    ]

    def fill_zeros(i, carry):
        for j in range(D // 16):
            zsum_v[i, pl.ds(j * 16, 16)] = jnp.zeros((16,), jnp.float32)
        return carry

    lax.fori_loop(0, RPW, fill_zeros, 0)

    pltpu.sync_copy(zsum_v, sums_sh.at[pl.ds(s * RPW, RPW)])

    for dma in b_dmas:
        dma.wait()

    @pl.when(wid == NW - 1)
    def _():
        for k in range(OVERLAP // 16):
            idx_v[k // (SUB // 16), pl.ds((k % (SUB // 16)) * 16, 16)] = (
                jnp.full((16,), B, jnp.int32))

    plsc.subcore_barrier()
    x_dma.wait()

    for j in range(NSUB):
        pltpu.sync_copy(rows_v.at[pl.ds(j * SUB, SUB)],
                        sums_sh.at[idx_v.at[j]], add=True)

    plsc.subcore_barrier()

    pltpu.sync_copy(sums_sh.at[pl.ds(s * RPW, RPW)],
                    sums_hbm.at[c, pl.ds(s * RPW, RPW)])


@functools.cache
def _sc_pool():
    mesh = plsc.VectorSubcoreMesh(core_axis_name="c", subcore_axis_name="s",
                                  num_cores=NC, num_subcores=NS)
    return pl.kernel(
        _sc_pool_body,
        out_type=jax.ShapeDtypeStruct((NC, R, D), jnp.float32),
        mesh=mesh,
        scratch_types=[
            pltpu.VMEM((CHUNK, D), jnp.float32),
            pltpu.VMEM((NSUB, SUB), jnp.int32),
            pltpu.VMEM((RPW, D), jnp.float32),
            pltpu.VMEM_SHARED((R, D), jnp.float32),
            pltpu.SemaphoreType.DMA,
            pltpu.SemaphoreType.DMA,
        ],
    )


def _mlp_body(sums_ref, idr_ref, idc_ref, w1_ref, b1_ref, w2_ref, b2_ref,
              out_ref):
    idr = idr_ref[...]
    idc = idc_ref[...]
    hi_ohT = (lax.broadcasted_iota(jnp.int32, (32, 1), 0)
              == idr // 16).astype(jnp.float32)
    lo_oh = (idc - (idc // 16) * 16
             == lax.broadcasted_iota(jnp.int32, (1, 16), 1)).astype(
        jnp.float32)
    grid = jnp.dot(hi_ohT, lo_oh,
                   preferred_element_type=jnp.float32)
    bidx = lax.broadcasted_iota(jnp.int32, (B, 1), 0)
    sel_h = (bidx // 16 == lax.broadcasted_iota(jnp.int32, (1, 32), 1)).astype(
        jnp.float32)
    sel_l = (bidx - (bidx // 16) * 16
             == lax.broadcasted_iota(jnp.int32, (1, 16), 1)).astype(
        jnp.float32)
    cnt = jnp.sum(
        jnp.dot(sel_h, grid, preferred_element_type=jnp.float32) * sel_l,
        axis=1, keepdims=True)

    sums = (sums_ref[0] + sums_ref[1])[:B]
    g = jnp.where(cnt > 0, sums / jnp.maximum(cnt, 1.0), 0.0)
    h = jnp.maximum(
        jnp.dot(g, w1_ref[...], preferred_element_type=jnp.float32)
        + b1_ref[...], 0.0)
    out_ref[...] = (
        jnp.dot(h, w2_ref[...], preferred_element_type=jnp.float32)
        + b2_ref[...])


def kernel(x, edge_index, batch, W1, b1, W2, b2):
    del edge_index
    sums = _sc_pool()(x, batch)
    ids = jnp.concatenate([batch, jnp.full((NP - N,), B, jnp.int32)])
    return pl.pallas_call(
        _mlp_body,
        out_shape=jax.ShapeDtypeStruct((B, C), jnp.float32),
    )(sums, ids.reshape(1, NP), ids.reshape(NP, 1),
      W1, b1.reshape(1, HID), W2, b2.reshape(1, C))


# speedup vs baseline: 1.2353x; 1.2353x over previous
import functools

import jax
import jax.numpy as jnp
from jax import lax
from jax.experimental import pallas as pl
from jax.experimental.pallas import tpu as pltpu
from jax.experimental.pallas import tpu_sc as plsc

N = 10000
D = 128
B = 512
HID = 64
C = 2

NC = 2
NS = 16
NW = NC * NS
CHUNK = 320
SUB = 64
NSUB = CHUNK // SUB
R = 640
RPW = R // NS
OVERLAP = NW * CHUNK - N

NP = NW * CHUNK
NPR = NP // 128

def _sc_pool_body(x_hbm, batch_hbm, sums_hbm,
                  rows_v, idx_v, zsum_v, sums_sh, xsem, bsem):
    c = lax.axis_index("c")
    s = lax.axis_index("s")
    wid = c * NS + s
    base = jnp.minimum(wid * CHUNK, N - CHUNK)

    x_dma = pltpu.async_copy(x_hbm.at[pl.ds(base, CHUNK)], rows_v, xsem)
    b_dmas = [
        pltpu.async_copy(batch_hbm.at[pl.ds(base + j * SUB, SUB)],
                         idx_v.at[j], bsem)
        for j in range(NSUB)
    ]

    def fill_zeros(i, carry):
        for j in range(D // 16):
            zsum_v[i, pl.ds(j * 16, 16)] = jnp.zeros((16,), jnp.float32)
        return carry

    lax.fori_loop(0, RPW, fill_zeros, 0)

    pltpu.sync_copy(zsum_v, sums_sh.at[pl.ds(s * RPW, RPW)])

    for dma in b_dmas:
        dma.wait()

    @pl.when(wid == NW - 1)
    def _():
        for k in range(OVERLAP // 16):
            idx_v[k // (SUB // 16), pl.ds((k % (SUB // 16)) * 16, 16)] = (
                jnp.full((16,), B, jnp.int32))

    plsc.subcore_barrier()
    x_dma.wait()

    for j in range(NSUB):
        pltpu.sync_copy(rows_v.at[pl.ds(j * SUB, SUB)],
                        sums_sh.at[idx_v.at[j]], add=True)

    plsc.subcore_barrier()

    pltpu.sync_copy(sums_sh.at[pl.ds(s * RPW, RPW)],
                    sums_hbm.at[c, pl.ds(s * RPW, RPW)])


@functools.cache
def _sc_pool():
    mesh = plsc.VectorSubcoreMesh(core_axis_name="c", subcore_axis_name="s",
                                  num_cores=NC, num_subcores=NS)
    return pl.kernel(
        _sc_pool_body,
        out_type=jax.ShapeDtypeStruct((NC, R, D), jnp.float32),
        mesh=mesh,
        scratch_types=[
            pltpu.VMEM((CHUNK, D), jnp.float32),
            pltpu.VMEM((NSUB, SUB), jnp.int32),
            pltpu.VMEM((RPW, D), jnp.float32),
            pltpu.VMEM_SHARED((R, D), jnp.float32),
            pltpu.SemaphoreType.DMA,
            pltpu.SemaphoreType.DMA,
        ],
    )


def _hist_body(idr_ref, cnt_ref):
    idr = idr_ref[...]
    hi_ohT = (lax.broadcasted_iota(jnp.int32, (32, 1), 0)
              == idr // 16).astype(jnp.float32)
    lo_ohT = (lax.broadcasted_iota(jnp.int32, (16, 1), 0)
              == idr - (idr // 16) * 16).astype(jnp.float32)
    grid = lax.dot_general(hi_ohT, lo_ohT, (((1,), (1,)), ((), ())),
                           preferred_element_type=jnp.float32)
    bidx = lax.broadcasted_iota(jnp.int32, (B, 1), 0)
    sel_h = (bidx // 16 == lax.broadcasted_iota(jnp.int32, (1, 32), 1)).astype(
        jnp.float32)
    sel_l = (bidx - (bidx // 16) * 16
             == lax.broadcasted_iota(jnp.int32, (1, 16), 1)).astype(
        jnp.float32)
    cnt_ref[...] = jnp.sum(
        jnp.dot(sel_h, grid, preferred_element_type=jnp.float32) * sel_l,
        axis=1, keepdims=True)


def _mlp_body(sums_ref, cnt_ref, w1_ref, b1_ref, w2_ref, b2_ref, out_ref):
    cnt = cnt_ref[...]
    sums = (sums_ref[0] + sums_ref[1])[:B]
    g = jnp.where(cnt > 0, sums / jnp.maximum(cnt, 1.0), 0.0)
    h = jnp.maximum(
        jnp.dot(g, w1_ref[...], preferred_element_type=jnp.float32)
        + b1_ref[...], 0.0)
    out_ref[...] = (
        jnp.dot(h, w2_ref[...], preferred_element_type=jnp.float32)
        + b2_ref[...])


def kernel(x, edge_index, batch, W1, b1, W2, b2):
    del edge_index
    ids = jnp.concatenate([batch, jnp.full((NP - N,), B, jnp.int32)])
    cnt = pl.pallas_call(
        _hist_body,
        out_shape=jax.ShapeDtypeStruct((B, 1), jnp.float32),
    )(ids.reshape(1, NP))
    sums = _sc_pool()(x, batch)
    return pl.pallas_call(
        _mlp_body,
        out_shape=jax.ShapeDtypeStruct((B, C), jnp.float32),
    )(sums, cnt, W1, b1.reshape(1, HID), W2, b2.reshape(1, C))


# speedup vs baseline: 1.2648x; 1.0239x over previous
import functools

import jax
import jax.numpy as jnp
from jax import lax
from jax.experimental import pallas as pl
from jax.experimental.pallas import tpu as pltpu
from jax.experimental.pallas import tpu_sc as plsc

N = 10000
D = 128
B = 512
HID = 64
C = 2

NC = 2
NS = 16
NW = NC * NS
CHUNK = 320
SUB = 80
NSUB = CHUNK // SUB
R = 640
RPW = R // NS
WB = 520 // RPW
OVERLAP = NW * CHUNK - N

NP = NW * CHUNK
NPR = NP // 128

def _sc_pool_body(x_hbm, batch_hbm, sums_hbm,
                  rows_v, idx_v, zsum_v, sums_sh, xsem, bsem):
    c = lax.axis_index("c")
    s = lax.axis_index("s")
    wid = c * NS + s
    base = jnp.minimum(wid * CHUNK, N - CHUNK)

    x_dmas = [
        pltpu.async_copy(x_hbm.at[pl.ds(base + j * SUB, SUB)],
                         rows_v.at[pl.ds(j * SUB, SUB)], xsem)
        for j in range(NSUB)
    ]
    b_dmas = [
        pltpu.async_copy(batch_hbm.at[pl.ds(base + j * SUB, SUB)],
                         idx_v.at[j], bsem)
        for j in range(NSUB)
    ]

    def fill_zeros(i, carry):
        for j in range(D // 16):
            zsum_v[i, pl.ds(j * 16, 16)] = jnp.zeros((16,), jnp.float32)
        return carry

    lax.fori_loop(0, RPW, fill_zeros, 0)

    pltpu.sync_copy(zsum_v, sums_sh.at[pl.ds(s * RPW, RPW)])

    for dma in b_dmas:
        dma.wait()

    @pl.when(wid == NW - 1)
    def _():
        for k in range(OVERLAP // 16):
            idx_v[k // (SUB // 16), pl.ds((k % (SUB // 16)) * 16, 16)] = (
                jnp.full((16,), B, jnp.int32))

    plsc.subcore_barrier()

    for j in range(NSUB):
        x_dmas[j].wait()
        pltpu.sync_copy(rows_v.at[pl.ds(j * SUB, SUB)],
                        sums_sh.at[idx_v.at[j]], add=True)

    plsc.subcore_barrier()

    @pl.when(s < WB)
    def _():
        pltpu.sync_copy(sums_sh.at[pl.ds(s * RPW, RPW)],
                        sums_hbm.at[c, pl.ds(s * RPW, RPW)])


@functools.cache
def _sc_pool():
    mesh = plsc.VectorSubcoreMesh(core_axis_name="c", subcore_axis_name="s",
                                  num_cores=NC, num_subcores=NS)
    return pl.kernel(
        _sc_pool_body,
        out_type=jax.ShapeDtypeStruct((NC, R, D), jnp.float32),
        mesh=mesh,
        scratch_types=[
            pltpu.VMEM((CHUNK, D), jnp.float32),
            pltpu.VMEM((NSUB, SUB), jnp.int32),
            pltpu.VMEM((RPW, D), jnp.float32),
            pltpu.VMEM_SHARED((R, D), jnp.float32),
            pltpu.SemaphoreType.DMA,
            pltpu.SemaphoreType.DMA,
        ],
    )


def _hist_body(idr_ref, cnt_ref):
    idr = idr_ref[...]
    hi_ohT = (lax.broadcasted_iota(jnp.int32, (32, 1), 0)
              == idr // 16).astype(jnp.float32)
    lo_ohT = (lax.broadcasted_iota(jnp.int32, (16, 1), 0)
              == idr - (idr // 16) * 16).astype(jnp.float32)
    grid = lax.dot_general(hi_ohT, lo_ohT, (((1,), (1,)), ((), ())),
                           preferred_element_type=jnp.float32)
    bidx = lax.broadcasted_iota(jnp.int32, (B, 1), 0)
    sel_h = (bidx // 16 == lax.broadcasted_iota(jnp.int32, (1, 32), 1)).astype(
        jnp.float32)
    sel_l = (bidx - (bidx // 16) * 16
             == lax.broadcasted_iota(jnp.int32, (1, 16), 1)).astype(
        jnp.float32)
    cnt_ref[...] = jnp.sum(
        jnp.dot(sel_h, grid, preferred_element_type=jnp.float32) * sel_l,
        axis=1, keepdims=True)


def _mlp_body(sums_ref, cnt_ref, w1_ref, b1_ref, w2_ref, b2_ref, out_ref):
    cnt = cnt_ref[...]
    sums = (sums_ref[0] + sums_ref[1])[:B]
    g = jnp.where(cnt > 0, sums / jnp.maximum(cnt, 1.0), 0.0)
    h = jnp.maximum(
        jnp.dot(g, w1_ref[...], preferred_element_type=jnp.float32)
        + b1_ref[...], 0.0)
    out_ref[...] = (
        jnp.dot(h, w2_ref[...], preferred_element_type=jnp.float32)
        + b2_ref[...])


def kernel(x, edge_index, batch, W1, b1, W2, b2):
    del edge_index
    ids = jnp.concatenate([batch, jnp.full((NP - N,), B, jnp.int32)])
    cnt = pl.pallas_call(
        _hist_body,
        out_shape=jax.ShapeDtypeStruct((B, 1), jnp.float32),
    )(ids.reshape(1, NP))
    sums = _sc_pool()(x, batch)
    return pl.pallas_call(
        _mlp_body,
        out_shape=jax.ShapeDtypeStruct((B, C), jnp.float32),
    )(sums, cnt, W1, b1.reshape(1, HID), W2, b2.reshape(1, C))


# speedup vs baseline: 1.2675x; 1.0021x over previous
import functools

import jax
import jax.numpy as jnp
from jax import lax
from jax.experimental import pallas as pl
from jax.experimental.pallas import tpu as pltpu
from jax.experimental.pallas import tpu_sc as plsc

N = 10000
D = 128
B = 512
HID = 64
C = 2

NC = 2
NS = 16
NW = NC * NS
CHUNK = 320
SUB = 80
NSUB = CHUNK // SUB
R = 640
RPW = R // NS
WB = 520 // RPW
OVERLAP = NW * CHUNK - N

NP = NW * CHUNK
NPR = NP // 128

def _sc_pool_body(x_hbm, batch_hbm, sums_hbm,
                  rows_v, idx_v, zsum_v, sums_sh, xsem, bsem):
    c = lax.axis_index("c")
    s = lax.axis_index("s")
    wid = c * NS + s
    base = jnp.minimum(wid * CHUNK, N - CHUNK)

    def start_stage(j, carry):
        pltpu.async_copy(x_hbm.at[pl.ds(base + j * SUB, SUB)],
                         rows_v.at[pl.ds(j * SUB, SUB)], xsem)
        pltpu.async_copy(batch_hbm.at[pl.ds(base + j * SUB, SUB)],
                         idx_v.at[j], bsem)
        return carry

    lax.fori_loop(0, NSUB, start_stage, 0)

    def fill_zeros(i, carry):
        for j in range(D // 16):
            zsum_v[i, pl.ds(j * 16, 16)] = jnp.zeros((16,), jnp.float32)
        return carry

    lax.fori_loop(0, RPW, fill_zeros, 0)

    pltpu.sync_copy(zsum_v, sums_sh.at[pl.ds(s * RPW, RPW)])

    def wait_batch(j, carry):
        pltpu.make_async_copy(batch_hbm.at[pl.ds(base + j * SUB, SUB)],
                              idx_v.at[j], bsem).wait()
        return carry

    lax.fori_loop(0, NSUB, wait_batch, 0)

    @pl.when(wid == NW - 1)
    def _():
        def patch(k, carry):
            idx_v[k // (SUB // 16), pl.ds((k % (SUB // 16)) * 16, 16)] = (
                jnp.full((16,), B, jnp.int32))
            return carry

        lax.fori_loop(0, OVERLAP // 16, patch, 0)

    plsc.subcore_barrier()

    def scat(j, carry):
        pltpu.make_async_copy(x_hbm.at[pl.ds(base + j * SUB, SUB)],
                              rows_v.at[pl.ds(j * SUB, SUB)], xsem).wait()
        pltpu.sync_copy(rows_v.at[pl.ds(j * SUB, SUB)],
                        sums_sh.at[idx_v.at[j]], add=True)
        return carry

    lax.fori_loop(0, NSUB, scat, 0)

    plsc.subcore_barrier()

    @pl.when(s < WB)
    def _():
        pltpu.sync_copy(sums_sh.at[pl.ds(s * RPW, RPW)],
                        sums_hbm.at[c, pl.ds(s * RPW, RPW)])


@functools.cache
def _sc_pool():
    mesh = plsc.VectorSubcoreMesh(core_axis_name="c", subcore_axis_name="s",
                                  num_cores=NC, num_subcores=NS)
    return pl.kernel(
        _sc_pool_body,
        out_type=jax.ShapeDtypeStruct((NC, R, D), jnp.float32),
        mesh=mesh,
        scratch_types=[
            pltpu.VMEM((CHUNK, D), jnp.float32),
            pltpu.VMEM((NSUB, SUB), jnp.int32),
            pltpu.VMEM((RPW, D), jnp.float32),
            pltpu.VMEM_SHARED((R, D), jnp.float32),
            pltpu.SemaphoreType.DMA,
            pltpu.SemaphoreType.DMA,
        ],
    )


def _hist_body(idr_ref, cnt_ref):
    idr = idr_ref[...]
    hi_ohT = (lax.broadcasted_iota(jnp.int32, (32, 1), 0)
              == idr // 16).astype(jnp.float32)
    lo_ohT = (lax.broadcasted_iota(jnp.int32, (16, 1), 0)
              == idr - (idr // 16) * 16).astype(jnp.float32)
    grid = lax.dot_general(hi_ohT, lo_ohT, (((1,), (1,)), ((), ())),
                           preferred_element_type=jnp.float32)
    bidx = lax.broadcasted_iota(jnp.int32, (B, 1), 0)
    sel_h = (bidx // 16 == lax.broadcasted_iota(jnp.int32, (1, 32), 1)).astype(
        jnp.float32)
    sel_l = (bidx - (bidx // 16) * 16
             == lax.broadcasted_iota(jnp.int32, (1, 16), 1)).astype(
        jnp.float32)
    cnt_ref[...] = jnp.sum(
        jnp.dot(sel_h, grid, preferred_element_type=jnp.float32) * sel_l,
        axis=1, keepdims=True)


def _mlp_body(sums_ref, cnt_ref, w1_ref, b1_ref, w2_ref, b2_ref, out_ref):
    cnt = cnt_ref[...]
    sums = (sums_ref[0] + sums_ref[1])[:B]
    g = jnp.where(cnt > 0, sums / jnp.maximum(cnt, 1.0), 0.0)
    h = jnp.maximum(
        jnp.dot(g, w1_ref[...], preferred_element_type=jnp.float32)
        + b1_ref[...], 0.0)
    out_ref[...] = (
        jnp.dot(h, w2_ref[...], preferred_element_type=jnp.float32)
        + b2_ref[...])


def kernel(x, edge_index, batch, W1, b1, W2, b2):
    del edge_index
    ids = jnp.concatenate([batch, jnp.full((NP - N,), B, jnp.int32)])
    cnt = pl.pallas_call(
        _hist_body,
        out_shape=jax.ShapeDtypeStruct((B, 1), jnp.float32),
    )(ids.reshape(1, NP))
    sums = _sc_pool()(x, batch)
    return pl.pallas_call(
        _mlp_body,
        out_shape=jax.ShapeDtypeStruct((B, C), jnp.float32),
    )(sums, cnt, W1, b1.reshape(1, HID), W2, b2.reshape(1, C))
